# Initial kernel scaffold; baseline (speedup 1.0000x reference)
import jax
import jax.numpy as jnp
from jax.experimental import pallas as pl


def kernel(static_input, table):
    raise NotImplementedError("write your pallas kernel here")


# speedup vs baseline: 1.9982x; 1.9982x over previous
import functools

import jax
import jax.numpy as jnp
from jax import lax
from jax.experimental import pallas as pl
from jax.experimental.pallas import tpu as pltpu
from jax.experimental.pallas import tpu_sc as plsc

_STATIC_VARIABLES = 26
_EMBEDDING_DIM = 64
_BATCH = 4096
_TOTAL = _BATCH * _STATIC_VARIABLES
_NC = 2
_NS = 16
_NW = _NC * _NS
_CHUNK = 128
_NCHUNK = _TOTAL // (_NW * _CHUNK)

_mesh = plsc.VectorSubcoreMesh(core_axis_name="c", subcore_axis_name="s")


@functools.partial(
    pl.kernel,
    mesh=_mesh,
    out_type=jax.ShapeDtypeStruct((_TOTAL, _EMBEDDING_DIM), jnp.float32),
    scratch_types=[
        pltpu.VMEM((_NCHUNK, _CHUNK), jnp.int32),
        pltpu.VMEM((_CHUNK, _EMBEDDING_DIM), jnp.float32),
        pltpu.SemaphoreType.DMA,
    ],
    compiler_params=pltpu.CompilerParams(use_tc_tiling_on_sc=False),
)
def _emb_lookup(idx_hbm, table_hbm, out_hbm, idx_v, rows_v, sem):
    wid = lax.axis_index("s") * _NC + lax.axis_index("c")
    chunk0 = wid * _NCHUNK
    pltpu.sync_copy(idx_hbm.at[wid], idx_v)

    def body(j, carry):
        pltpu.async_copy(table_hbm.at[idx_v.at[j]], rows_v, sem).wait()
        pltpu.sync_copy(rows_v, out_hbm.at[pl.ds((chunk0 + j) * _CHUNK, _CHUNK)])
        return carry

    lax.fori_loop(0, _NCHUNK, body, 0)


def kernel(static_input, table):
    idx = static_input.reshape(_NW, _NCHUNK, _CHUNK).astype(jnp.int32)
    out = _emb_lookup(idx, table.astype(jnp.float32))
    return out.reshape(_BATCH, _STATIC_VARIABLES * _EMBEDDING_DIM)


# speedup vs baseline: 2.0663x; 1.0341x over previous
import functools

import jax
import jax.numpy as jnp
from jax import lax
from jax.experimental import pallas as pl
from jax.experimental.pallas import tpu as pltpu
from jax.experimental.pallas import tpu_sc as plsc

_STATIC_VARIABLES = 26
_EMBEDDING_DIM = 64
_BATCH = 4096
_TOTAL = _BATCH * _STATIC_VARIABLES
_NC = 2
_NS = 16
_NW = _NC * _NS
_CHUNK = 416
_NCHUNK = _TOTAL // (_NW * _CHUNK)

_mesh = plsc.VectorSubcoreMesh(core_axis_name="c", subcore_axis_name="s")


@functools.partial(
    pl.kernel,
    mesh=_mesh,
    out_type=jax.ShapeDtypeStruct((_TOTAL, _EMBEDDING_DIM), jnp.float32),
    scratch_types=[
        pltpu.VMEM((_NCHUNK, _CHUNK), jnp.int32),
        pltpu.VMEM((_CHUNK, _EMBEDDING_DIM), jnp.float32),
        pltpu.VMEM((_CHUNK, _EMBEDDING_DIM), jnp.float32),
        pltpu.SemaphoreType.DMA,
        pltpu.SemaphoreType.DMA,
        pltpu.SemaphoreType.DMA,
        pltpu.SemaphoreType.DMA,
    ],
    compiler_params=pltpu.CompilerParams(use_tc_tiling_on_sc=False),
)
def _emb_lookup(idx_hbm, table_hbm, out_hbm, idx_v, rows0, rows1, g0, g1, w0, w1):
    wid = lax.axis_index("s") * _NC + lax.axis_index("c")
    chunk0 = wid * _NCHUNK
    pltpu.sync_copy(idx_hbm.at[wid], idx_v)

    bufs = (rows0, rows1)
    gsems = (g0, g1)
    wsems = (w0, w1)

    def gather(j):
        b = j % 2
        return pltpu.async_copy(table_hbm.at[idx_v.at[j]], bufs[b], gsems[b])

    gathers = [None] * _NCHUNK
    writes = [None] * _NCHUNK
    gathers[0] = gather(0)
    for j in range(_NCHUNK):
        b = j % 2
        if j + 1 < _NCHUNK:
            if j >= 1:
                writes[j - 1].wait()
            gathers[j + 1] = gather(j + 1)
        gathers[j].wait()
        writes[j] = pltpu.async_copy(
            bufs[b], out_hbm.at[pl.ds((chunk0 + j) * _CHUNK, _CHUNK)], wsems[b]
        )
    writes[_NCHUNK - 2].wait()
    writes[_NCHUNK - 1].wait()


def kernel(static_input, table):
    idx = static_input.reshape(_NW, _NCHUNK, _CHUNK).astype(jnp.int32)
    out = _emb_lookup(idx, table.astype(jnp.float32))
    return out.reshape(_BATCH, _STATIC_VARIABLES * _EMBEDDING_DIM)


# speedup vs baseline: 8.0374x; 3.8899x over previous
import functools

import jax
import jax.numpy as jnp
from jax import lax
from jax.experimental import pallas as pl
from jax.experimental.pallas import tpu as pltpu
from jax.experimental.pallas import tpu_sc as plsc

_STATIC_VARIABLES = 26
_EMBEDDING_DIM = 64
_BATCH = 4096
_TOTAL = _BATCH * _STATIC_VARIABLES
_NC = 2
_NS = 16
_NW = _NC * _NS
_CHUNK = 416
_NCHUNK = _TOTAL // (_NW * _CHUNK)

_mesh = plsc.VectorSubcoreMesh(core_axis_name="c", subcore_axis_name="s")


@functools.partial(
    pl.kernel,
    mesh=_mesh,
    out_type=jax.ShapeDtypeStruct((_TOTAL, _EMBEDDING_DIM), jnp.float32),
    scratch_types=[
        pltpu.VMEM((_NCHUNK, _CHUNK), jnp.int32),
        pltpu.VMEM((_CHUNK, _EMBEDDING_DIM), jnp.float32),
        pltpu.VMEM((_CHUNK, _EMBEDDING_DIM), jnp.float32),
        pltpu.SemaphoreType.DMA,
        pltpu.SemaphoreType.DMA,
        pltpu.SemaphoreType.DMA,
        pltpu.SemaphoreType.DMA,
        pltpu.VMEM_SHARED((_STATIC_VARIABLES, _EMBEDDING_DIM), jnp.float32),
    ],
    compiler_params=pltpu.CompilerParams(use_tc_tiling_on_sc=False),
)
def _emb_lookup(idx_hbm, table_hbm, out_hbm, idx_v, rows0, rows1, g0, g1, w0, w1, tab_sh):
    sid = lax.axis_index("s")
    wid = sid * _NC + lax.axis_index("c")
    chunk0 = wid * _NCHUNK
    @pl.when(sid == 0)
    def _():
        pltpu.sync_copy(table_hbm, tab_sh)

    pltpu.sync_copy(idx_hbm.at[wid], idx_v)
    plsc.subcore_barrier()

    bufs = (rows0, rows1)
    gsems = (g0, g1)
    wsems = (w0, w1)

    def gather(j):
        b = j % 2
        return pltpu.async_copy(tab_sh.at[idx_v.at[j]], bufs[b], gsems[b])

    gathers = [None] * _NCHUNK
    writes = [None] * _NCHUNK
    gathers[0] = gather(0)
    for j in range(_NCHUNK):
        b = j % 2
        if j + 1 < _NCHUNK:
            if j >= 1:
                writes[j - 1].wait()
            gathers[j + 1] = gather(j + 1)
        gathers[j].wait()
        writes[j] = pltpu.async_copy(
            bufs[b], out_hbm.at[pl.ds((chunk0 + j) * _CHUNK, _CHUNK)], wsems[b]
        )
    writes[_NCHUNK - 2].wait()
    writes[_NCHUNK - 1].wait()


def kernel(static_input, table):
    idx = static_input.reshape(_NW, _NCHUNK, _CHUNK).astype(jnp.int32)
    out = _emb_lookup(idx, table.astype(jnp.float32))
    return out.reshape(_BATCH, _STATIC_VARIABLES * _EMBEDDING_DIM)
